# baseline (speedup 1.0000x reference)
import functools

import jax
import jax.numpy as jnp
from jax import lax
from jax.experimental import pallas as pl
from jax.experimental.pallas import tpu as pltpu
from jax.experimental.pallas import tpu_sc as plsc

B = 8
H = 16
S = 2048
D = 128
ROWS = B * H * S
NW = 32
GROUPS_PER_W = (B * H) // NW
BPC = B // 2
L = 16


def _sc_body(pos_ref, kv_ref, vv_ref, kc_ref, vc_ref, ko_ref, vo_ref,
             pos_v, sidx_v, didx_v, kstage_v, vstage_v, sem):
    c = lax.axis_index("c")
    s = lax.axis_index("s")
    w = c * 16 + s

    copies = []
    for j in range(GROUPS_PER_W):
        r = (w * GROUPS_PER_W + j) * S
        copies.append(pltpu.async_copy(kc_ref.at[pl.ds(r, S)],
                                       ko_ref.at[pl.ds(r, S)], sem))
        copies.append(pltpu.async_copy(vc_ref.at[pl.ds(r, S)],
                                       vo_ref.at[pl.ds(r, S)], sem))
    for cp in copies:
        cp.wait()
    plsc.subcore_barrier()

    pltpu.sync_copy(pos_ref.at[c], pos_v)
    pv = pos_v[...]
    bl = c * BPC + lax.iota(jnp.int32, L) % BPC
    sidx_v[...] = bl * H + s
    didx_v[...] = (bl * H + pv - 1) * S + s
    kg = pltpu.async_copy(kv_ref.at[sidx_v], kstage_v, sem)
    vg = pltpu.async_copy(vv_ref.at[sidx_v], vstage_v, sem)
    kg.wait()
    vg.wait()
    ks = pltpu.async_copy(kstage_v, ko_ref.at[didx_v], sem)
    vs = pltpu.async_copy(vstage_v, vo_ref.at[didx_v], sem)
    ks.wait()
    vs.wait()


def kernel(input_pos, k_val, v_val, k_cache, v_cache):
    posA = jnp.tile(input_pos.reshape(2, BPC), (1, L // BPC))
    kv2 = k_val.reshape(B * H, D)
    vv2 = v_val.reshape(B * H, D)
    kc2 = k_cache.reshape(ROWS, D)
    vc2 = v_cache.reshape(ROWS, D)

    mesh = plsc.VectorSubcoreMesh(core_axis_name="c", subcore_axis_name="s",
                                  num_cores=2)
    run = functools.partial(
        pl.kernel,
        out_type=[
            jax.ShapeDtypeStruct((ROWS, D), jnp.float32),
            jax.ShapeDtypeStruct((ROWS, D), jnp.float32),
        ],
        mesh=mesh,
        scratch_types=[
            pltpu.VMEM((L,), jnp.int32),
            pltpu.VMEM((L,), jnp.int32),
            pltpu.VMEM((L,), jnp.int32),
            pltpu.VMEM((L, D), jnp.float32),
            pltpu.VMEM((L, D), jnp.float32),
            pltpu.SemaphoreType.DMA,
        ],
    )(_sc_body)
    k2, v2 = run(posA, kv2, vv2, kc2, vc2)
    return (k2.reshape(B, H, S, D), v2.reshape(B, H, S, D))


# speedup vs baseline: 48.0318x; 48.0318x over previous
import jax
import jax.numpy as jnp
from jax.experimental import pallas as pl
from jax.experimental.pallas import tpu as pltpu

B = 8
H = 16
S = 2048
D = 128
HB = 2


def _body(pos_ref, kc_ref, vc_ref, kv_ref, vv_ref, ko_ref, vo_ref):
    b = pl.program_id(0)
    h = pl.program_id(1)
    ko_ref[...] = kc_ref[...]
    vo_ref[...] = vc_ref[...]
    hh = pos_ref[b] - 1

    @pl.when(hh // HB == h)
    def _():
        ko_ref[0, hh % HB, 0:16, :] = kv_ref[0, 0, :, :]
        vo_ref[0, hh % HB, 0:16, :] = vv_ref[0, 0, :, :]


def kernel(input_pos, k_val, v_val, k_cache, v_cache):
    grid_spec = pltpu.PrefetchScalarGridSpec(
        num_scalar_prefetch=1,
        grid=(B, H // HB),
        in_specs=[
            pl.BlockSpec((1, HB, S, D), lambda b, h, pos: (b, h, 0, 0)),
            pl.BlockSpec((1, HB, S, D), lambda b, h, pos: (b, h, 0, 0)),
            pl.BlockSpec((1, 1, H, D), lambda b, h, pos: (b, 0, 0, 0)),
            pl.BlockSpec((1, 1, H, D), lambda b, h, pos: (b, 0, 0, 0)),
        ],
        out_specs=[
            pl.BlockSpec((1, HB, S, D), lambda b, h, pos: (b, h, 0, 0)),
            pl.BlockSpec((1, HB, S, D), lambda b, h, pos: (b, h, 0, 0)),
        ],
    )
    k_out, v_out = pl.pallas_call(
        _body,
        grid_spec=grid_spec,
        out_shape=[
            jax.ShapeDtypeStruct((B, H, S, D), jnp.float32),
            jax.ShapeDtypeStruct((B, H, S, D), jnp.float32),
        ],
    )(input_pos, k_cache, v_cache, k_val, v_val)
    return (k_out, v_out)


# speedup vs baseline: 48.7853x; 1.0157x over previous
import jax
import jax.numpy as jnp
from jax.experimental import pallas as pl
from jax.experimental.pallas import tpu as pltpu

B = 8
H = 16
S = 2048
D = 128
HB = 4


def _body(pos_ref, kc_ref, vc_ref, kv_ref, vv_ref, ko_ref, vo_ref):
    b = pl.program_id(0)
    h = pl.program_id(1)
    ko_ref[...] = kc_ref[...]
    vo_ref[...] = vc_ref[...]
    hh = pos_ref[b] - 1

    @pl.when(hh // HB == h)
    def _():
        ko_ref[0, hh % HB, 0:16, :] = kv_ref[0, 0, :, :]
        vo_ref[0, hh % HB, 0:16, :] = vv_ref[0, 0, :, :]


def kernel(input_pos, k_val, v_val, k_cache, v_cache):
    grid_spec = pltpu.PrefetchScalarGridSpec(
        num_scalar_prefetch=1,
        grid=(B, H // HB),
        in_specs=[
            pl.BlockSpec((1, HB, S, D), lambda b, h, pos: (b, h, 0, 0)),
            pl.BlockSpec((1, HB, S, D), lambda b, h, pos: (b, h, 0, 0)),
            pl.BlockSpec((1, 1, H, D), lambda b, h, pos: (b, 0, 0, 0)),
            pl.BlockSpec((1, 1, H, D), lambda b, h, pos: (b, 0, 0, 0)),
        ],
        out_specs=[
            pl.BlockSpec((1, HB, S, D), lambda b, h, pos: (b, h, 0, 0)),
            pl.BlockSpec((1, HB, S, D), lambda b, h, pos: (b, h, 0, 0)),
        ],
    )
    k_out, v_out = pl.pallas_call(
        _body,
        grid_spec=grid_spec,
        out_shape=[
            jax.ShapeDtypeStruct((B, H, S, D), jnp.float32),
            jax.ShapeDtypeStruct((B, H, S, D), jnp.float32),
        ],
    )(input_pos, k_cache, v_cache, k_val, v_val)
    return (k_out, v_out)
